# baseline (speedup 1.0000x reference)
import functools

import jax
import jax.numpy as jnp
from jax.experimental import pallas as pl


def _pick_rows(n, target):
    best = 8
    b = 8
    while b <= target:
        if n % b == 0:
            best = b
        b += 8
    return best


def _bdot(a, b):
    return jnp.dot(a.astype(jnp.bfloat16), b.astype(jnp.bfloat16),
                   preferred_element_type=jnp.float32)


def _mm_body(a_ref, h_ref, o_ref):
    o_ref[...] = _bdot(a_ref[...], h_ref[...])


def _adj_matmul(adj, h):
    n, k = adj.shape
    d = h.shape[1]
    bm = _pick_rows(n, 400)
    return pl.pallas_call(
        _mm_body,
        grid=(n // bm,),
        in_specs=[
            pl.BlockSpec((bm, k), lambda i: (i, 0)),
            pl.BlockSpec((k, d), lambda i: (0, 0)),
        ],
        out_specs=pl.BlockSpec((bm, d), lambda i: (i, 0)),
        out_shape=jax.ShapeDtypeStruct((n, d), jnp.float32),
    )(adj, h)


def _bn_relu(v, g, be):
    mean = jnp.mean(v, axis=0, keepdims=True)
    d = v - mean
    var = jnp.mean(d * d, axis=0, keepdims=True)
    return jnp.maximum(g * d / jnp.sqrt(var + 1e-5) + be, 0.0)


def _mlp_body(p_ref, w1_ref, b1_ref, g1_ref, be1_ref,
              w2_ref, b2_ref, g2_ref, be2_ref, o_ref):
    y = _bdot(p_ref[...], w1_ref[...]) + b1_ref[...]
    h1 = _bn_relu(y, g1_ref[...], be1_ref[...])
    z = _bdot(h1, w2_ref[...]) + b2_ref[...]
    o_ref[...] = _bn_relu(z, g2_ref[...], be2_ref[...]).astype(o_ref.dtype)


def _mlp_stage(pooled, p, out_dtype):
    n, d = pooled.shape
    hdim = p['W1'].shape[1]
    return pl.pallas_call(
        _mlp_body,
        out_shape=jax.ShapeDtypeStruct((n, hdim), out_dtype),
    )(pooled, p['W1'], _row(p['b1']), _row(p['g1']), _row(p['be1']),
      p['W2'], _row(p['b2']), _row(p['bn_g']), _row(p['bn_b']))


def _pool_body(gp_ref, h_ref, o_ref):
    o_ref[...] = _bdot(gp_ref[...], h_ref[...])


def _graph_pool_mm(graph_pool, h):
    g, n = graph_pool.shape
    d = h.shape[1]
    return pl.pallas_call(
        _pool_body,
        out_shape=jax.ShapeDtypeStruct((g, d), jnp.float32),
    )(graph_pool, h)


def _row(v):
    return v.reshape(1, -1)


def kernel(x, graph_pool, adj, params):
    h = x.astype(jnp.bfloat16)
    n_layers = len(params)
    for li, p in enumerate(params):
        pooled = _adj_matmul(adj, h)
        last = li == n_layers - 1
        h = _mlp_stage(pooled, p, jnp.float32 if last else jnp.bfloat16)
    h_nodes = h
    pooled_h = _graph_pool_mm(graph_pool, h_nodes)
    return (pooled_h, h_nodes)
